# baseline (speedup 1.0000x reference)
import numpy as np
import jax
import jax.numpy as jnp
from jax.experimental import pallas as pl
from jax.experimental.pallas import tpu as pltpu
from jax.experimental.pallas import tpu_sc as plsc

_N = 512
_D = 128
_H = 4
_F = 64
_E = _N * (_N - 1)

_SC_NC = 2
_SC_NS = 16
_SC_NW = _SC_NC * _SC_NS
_SC_CHUNK = 8192
_SC_LAST = _E - (_SC_NW - 1) * _SC_CHUNK


def _build_edge_index() -> np.ndarray:
    base = np.arange(_N - 1, dtype=np.int32)[None, :]
    src_col = np.arange(_N, dtype=np.int32)[:, None]
    dst = (base + (base >= src_col).astype(np.int32)).reshape(-1)
    src = np.repeat(np.arange(_N, dtype=np.int32), _N - 1)
    return np.stack([src, dst])


_EDGE_INDEX = _build_edge_index()


def _leaky(x):
    return jnp.maximum(x, 0.2 * x)


def _gat_adj_kernel(emb_ref, w_ref, asrc_ref, adst_ref, out_ref):
    emb = emb_ref[:]
    w = w_ref[:]
    row = jax.lax.broadcasted_iota(jnp.int32, (_N, _N), 0)
    col = jax.lax.broadcasted_iota(jnp.int32, (_N, _N), 1)
    diag = row == col

    hp = jax.lax.Precision.HIGHEST

    def dot_t(a, b):
        return jax.lax.dot_general(
            a, b, (((1,), (1,)), ((), ())),
            preferred_element_type=jnp.float32, precision=hp)

    gi = jax.lax.broadcasted_iota(jnp.int32, (_H, _H * _F), 1) // _F
    hi = jax.lax.broadcasted_iota(jnp.int32, (_H, _H * _F), 0)
    head_sel = (gi == hi).astype(jnp.float32)
    asrc_tiled = jnp.concatenate([asrc_ref[:]] * _H, axis=1)
    adst_tiled = jnp.concatenate([adst_ref[:]] * _H, axis=1)
    a_s = head_sel * asrc_tiled
    a_d = head_sel * adst_tiled
    ws_t = dot_t(a_s, w)
    wd_t = dot_t(a_d, w)
    s_rows = dot_t(ws_t, emb)
    d_rows = dot_t(wd_t, emb)
    s_cols = jax.lax.dot_general(
        emb, ws_t, (((1,), (1,)), ((), ())),
        preferred_element_type=jnp.float32, precision=hp)

    m1 = jnp.max(s_rows, axis=1, keepdims=True)
    eq = s_rows == m1
    n_eq = jnp.sum(eq.astype(jnp.float32), axis=1, keepdims=True)
    m2 = jnp.max(jnp.where(eq, -jnp.inf, s_rows), axis=1, keepdims=True)
    m_at_eq = jnp.where(n_eq > 1.0, m1, m2)
    s_noj = jnp.where(eq, m_at_eq, m1)
    amax_all = _leaky(s_noj + d_rows)

    acc = None
    for h in range(_H):
        s = s_cols[:, h:h + 1]
        d = d_rows[h:h + 1, :]
        amax = amax_all[h:h + 1, :]
        p = jnp.where(diag, 0.0, jnp.exp(_leaky(s + d) - amax))
        denom = jnp.sum(p, axis=0, keepdims=True) + 1e-16
        contrib = p * ((1.0 / _H) / denom)
        acc = contrib if acc is None else acc + contrib
    out_ref[:] = acc


def _sc_emit_edges_kernel(ei_hbm, out_hbm, buf):
    c = jax.lax.axis_index("c")
    s = jax.lax.axis_index("s")
    wid = s * _SC_NC + c
    base = wid * _SC_CHUNK

    @pl.when(wid != _SC_NW - 1)
    def _full():
        pltpu.sync_copy(ei_hbm.at[:, pl.ds(base, _SC_CHUNK)], buf)
        pltpu.sync_copy(buf, out_hbm.at[:, pl.ds(base, _SC_CHUNK)])

    @pl.when(wid == _SC_NW - 1)
    def _trimmed():
        pltpu.sync_copy(ei_hbm.at[:, pl.ds(base, _SC_LAST)],
                        buf.at[:, pl.ds(0, _SC_LAST)])
        pltpu.sync_copy(buf.at[:, pl.ds(0, _SC_LAST)],
                        out_hbm.at[:, pl.ds(base, _SC_LAST)])


_sc_emit_edges = pl.kernel(
    _sc_emit_edges_kernel,
    out_type=jax.ShapeDtypeStruct((2, _E), jnp.int32),
    mesh=plsc.VectorSubcoreMesh(core_axis_name="c", subcore_axis_name="s"),
    scratch_types=[pltpu.VMEM((2, _SC_CHUNK), jnp.int32)],
)


@jax.jit
def _gat(input_emb, W, att_src, att_dst):
    adj = pl.pallas_call(
        _gat_adj_kernel,
        out_shape=jax.ShapeDtypeStruct((_N, _N), jnp.float32),
    )(input_emb, W, att_src, att_dst)
    edge_index = _sc_emit_edges(jnp.asarray(_EDGE_INDEX))
    return adj, edge_index


def kernel(input_emb, W, att_src, att_dst, bias):
    del bias
    adj_matrix, edge_index = _gat(input_emb, W, att_src, att_dst)
    return (edge_index, adj_matrix)


# speedup vs baseline: 3.0784x; 3.0784x over previous
import numpy as np
import jax
import jax.numpy as jnp
from jax.experimental import pallas as pl
_N = 512
_D = 128
_H = 4
_F = 64


def _build_edge_index() -> np.ndarray:
    base = np.arange(_N - 1, dtype=np.int32)[None, :]
    src_col = np.arange(_N, dtype=np.int32)[:, None]
    dst = (base + (base >= src_col).astype(np.int32)).reshape(-1)
    src = np.repeat(np.arange(_N, dtype=np.int32), _N - 1)
    return np.stack([src, dst])


_EDGE_INDEX = _build_edge_index()


def _leaky(x):
    return jnp.maximum(x, 0.2 * x)


def _gat_adj_kernel(emb_ref, w_ref, asrc_ref, adst_ref, out_ref):
    emb = emb_ref[:]
    w = w_ref[:]
    row = jax.lax.broadcasted_iota(jnp.int32, (_N, 1), 0)
    col = jax.lax.broadcasted_iota(jnp.int32, (1, _N), 1)
    diag = row == col

    def dot_t(a, b):
        return jax.lax.dot_general(
            a, b, (((1,), (1,)), ((), ())),
            preferred_element_type=jnp.float32)

    def split3(x, lead):
        hi = x.astype(jnp.bfloat16)
        lo = (x - hi.astype(jnp.float32)).astype(jnp.bfloat16)
        parts = (hi, hi, lo) if lead else (hi, lo, hi)
        return jnp.concatenate(parts, axis=1)

    gi = jax.lax.broadcasted_iota(jnp.int32, (_H, _H * _F), 1) // _F
    hi = jax.lax.broadcasted_iota(jnp.int32, (_H, _H * _F), 0)
    head_sel = (gi == hi).astype(jnp.float32)
    asrc_tiled = jnp.concatenate([asrc_ref[:]] * _H, axis=1)
    adst_tiled = jnp.concatenate([adst_ref[:]] * _H, axis=1)
    a_s = head_sel * asrc_tiled
    a_d = head_sel * adst_tiled
    w3 = split3(w, lead=False)
    ws_t = dot_t(split3(a_s, lead=True), w3)
    wd_t = dot_t(split3(a_d, lead=True), w3)
    emb3 = split3(emb, lead=False)
    ws3 = split3(ws_t, lead=True)
    wd3 = split3(wd_t, lead=True)
    s_rows = dot_t(ws3, emb3)
    d_rows = dot_t(wd3, emb3)
    s_cols = jax.lax.dot_general(
        emb3, ws3, (((1,), (1,)), ((), ())),
        preferred_element_type=jnp.float32)

    m1 = jnp.max(s_rows, axis=1, keepdims=True)
    eq = s_rows == m1
    n_eq = jnp.sum(eq.astype(jnp.float32), axis=1, keepdims=True)
    m2 = jnp.max(jnp.where(eq, -jnp.inf, s_rows), axis=1, keepdims=True)
    m_at_eq = jnp.where(n_eq > 1.0, m1, m2)
    s_noj = jnp.where(eq, m_at_eq, m1)
    amax_all = _leaky(s_noj + d_rows)

    acc = None
    for h in range(_H):
        s = s_cols[:, h:h + 1]
        d = d_rows[h:h + 1, :]
        amax = amax_all[h:h + 1, :]
        p = jnp.where(diag, 0.0, jnp.exp(_leaky(s + d) - amax))
        denom = jnp.sum(p, axis=0, keepdims=True) + 1e-16
        contrib = p * ((1.0 / _H) / denom)
        acc = contrib if acc is None else acc + contrib
    out_ref[:] = acc


@jax.jit
def _gat(input_emb, W, att_src, att_dst):
    adj = pl.pallas_call(
        _gat_adj_kernel,
        out_shape=jax.ShapeDtypeStruct((_N, _N), jnp.float32),
    )(input_emb, W, att_src, att_dst)
    return adj, jnp.asarray(_EDGE_INDEX)


def kernel(input_emb, W, att_src, att_dst, bias):
    del bias
    adj_matrix, edge_index = _gat(input_emb, W, att_src, att_dst)
    return (edge_index, adj_matrix)


# speedup vs baseline: 3.1241x; 1.0149x over previous
import numpy as np
import jax
import jax.numpy as jnp
from jax.experimental import pallas as pl
_N = 512
_D = 128
_H = 4
_F = 64


def _build_edge_index() -> np.ndarray:
    base = np.arange(_N - 1, dtype=np.int32)[None, :]
    src_col = np.arange(_N, dtype=np.int32)[:, None]
    dst = (base + (base >= src_col).astype(np.int32)).reshape(-1)
    src = np.repeat(np.arange(_N, dtype=np.int32), _N - 1)
    return np.stack([src, dst])


_EDGE_INDEX = _build_edge_index()


def _leaky(x):
    return jnp.maximum(x, 0.2 * x)


def _gat_adj_kernel(emb_ref, w_ref, asrc_ref, adst_ref, out_ref):
    emb = emb_ref[:]
    w = w_ref[:]
    row = jax.lax.broadcasted_iota(jnp.int32, (_N, 1), 0)
    col = jax.lax.broadcasted_iota(jnp.int32, (1, _N), 1)
    diag = row == col

    def dot_t(a, b):
        return jax.lax.dot_general(
            a, b, (((1,), (1,)), ((), ())),
            preferred_element_type=jnp.float32)

    def split3(x, lead):
        hi = x.astype(jnp.bfloat16)
        lo = (x - hi.astype(jnp.float32)).astype(jnp.bfloat16)
        parts = (hi, hi, lo) if lead else (hi, lo, hi)
        return jnp.concatenate(parts, axis=1)

    gi = jax.lax.broadcasted_iota(jnp.int32, (_H, _H * _F), 1) // _F
    hi = jax.lax.broadcasted_iota(jnp.int32, (_H, _H * _F), 0)
    head_sel = (gi == hi).astype(jnp.float32)
    asrc_tiled = jnp.concatenate([asrc_ref[:]] * _H, axis=1)
    adst_tiled = jnp.concatenate([adst_ref[:]] * _H, axis=1)
    a_s = head_sel * asrc_tiled
    a_d = head_sel * adst_tiled
    w3 = split3(w, lead=False)
    ws_t = dot_t(split3(a_s, lead=True), w3)
    wd_t = dot_t(split3(a_d, lead=True), w3)
    emb3 = split3(emb, lead=False)
    ws3 = split3(ws_t, lead=True)
    wd3 = split3(wd_t, lead=True)
    s_rows = dot_t(ws3, emb3)
    d_rows = dot_t(wd3, emb3)
    s_cols = jax.lax.dot_general(
        emb3, ws3, (((1,), (1,)), ((), ())),
        preferred_element_type=jnp.float32)

    m1 = jnp.max(s_rows, axis=1, keepdims=True)
    eq = s_rows == m1
    n_eq = jnp.sum(eq.astype(jnp.float32), axis=1, keepdims=True)
    m2 = jnp.max(jnp.where(eq, -jnp.inf, s_rows), axis=1, keepdims=True)
    m_at_eq = jnp.where(n_eq > 1.0, m1, m2)
    s_noj = jnp.where(eq, m_at_eq, m1)
    amax_all = _leaky(s_noj + d_rows)

    d_hi = d_rows - amax_all
    d_lo = 0.2 * d_rows - amax_all
    s_lo_cols = 0.2 * s_cols

    acc = None
    for h in range(_H):
        s = s_cols[:, h:h + 1]
        s_lo = s_lo_cols[:, h:h + 1]
        u = jnp.maximum(s + d_hi[h:h + 1, :], s_lo + d_lo[h:h + 1, :])
        p = jnp.where(diag, 0.0, jnp.exp(u))
        denom = jnp.sum(p, axis=0, keepdims=True) + 1e-16
        contrib = p * ((1.0 / _H) / denom)
        acc = contrib if acc is None else acc + contrib
    out_ref[:] = acc


@jax.jit
def _gat(input_emb, W, att_src, att_dst):
    adj = pl.pallas_call(
        _gat_adj_kernel,
        out_shape=jax.ShapeDtypeStruct((_N, _N), jnp.float32),
    )(input_emb, W, att_src, att_dst)
    return adj, jnp.asarray(_EDGE_INDEX)


def kernel(input_emb, W, att_src, att_dst, bias):
    del bias
    adj_matrix, edge_index = _gat(input_emb, W, att_src, att_dst)
    return (edge_index, adj_matrix)
